# Initial kernel scaffold; baseline (speedup 1.0000x reference)
import jax
import jax.numpy as jnp
from jax.experimental import pallas as pl


def kernel(xyz, params):
    raise NotImplementedError("write your pallas kernel here")


# speedup vs baseline: 4.6935x; 4.6935x over previous
import functools
import math

import jax
import jax.numpy as jnp
from jax.experimental import pallas as pl

HIGH = jax.lax.Precision.HIGHEST
EPS = 1e-5


def _dotf(a, b):
    return jnp.dot(a, b, precision=HIGH, preferred_element_type=jnp.float32)


def _fps_body(x_ref, y_ref, z_ref, ox_ref, oy_ref, oz_ref, *, S, N, B):
    X = x_ref[...]
    Y = y_ref[...]
    Z = z_ref[...]
    lid = jax.lax.broadcasted_iota(jnp.int32, (B, N), 1)
    sid = jax.lax.broadcasted_iota(jnp.int32, (B, S), 1)

    def body(i, st):
        dist, oh, aX, aY, aZ = st
        cx = jnp.sum(X * oh, axis=1, keepdims=True)
        cy = jnp.sum(Y * oh, axis=1, keepdims=True)
        cz = jnp.sum(Z * oh, axis=1, keepdims=True)
        aX = jnp.where(sid == i, cx, aX)
        aY = jnp.where(sid == i, cy, aY)
        aZ = jnp.where(sid == i, cz, aZ)
        d = (X - cx) ** 2 + (Y - cy) ** 2 + (Z - cz) ** 2
        dist = jnp.minimum(dist, d)
        m = jnp.max(dist, axis=1, keepdims=True)
        cand = jnp.where(dist == m, lid, N)
        idx = jnp.min(cand, axis=1, keepdims=True)
        oh = (lid == idx).astype(jnp.float32)
        return dist, oh, aX, aY, aZ

    dist0 = jnp.full((B, N), 1e10, jnp.float32)
    oh0 = (lid == 0).astype(jnp.float32)
    z = jnp.zeros((B, S), jnp.float32)
    _, _, aX, aY, aZ = jax.lax.fori_loop(0, S, body, (dist0, oh0, z, z, z))
    ox_ref[...] = aX
    oy_ref[...] = aY
    oz_ref[...] = aZ


def _fps(X, Y, Z, S):
    B, N = X.shape
    out = jax.ShapeDtypeStruct((B, S), jnp.float32)
    return pl.pallas_call(
        functools.partial(_fps_body, S=S, N=N, B=B),
        out_shape=(out, out, out),
    )(X, Y, Z)


def _ball_sa_body(tab_ref, xr_ref, yr_ref, zr_ref, sx_ref, sy_ref, sz_ref,
                  L_ref, W1_ref, b1_ref, g1_ref, e1_ref, W2_ref, b2_ref,
                  g2_ref, e2_ref, W3_ref, b3_ref, g3_ref, e3_ref, out_ref,
                  *, Np, K, St, r2, Ct):
    Xr = xr_ref[0]
    Yr = yr_ref[0]
    Zr = zr_ref[0]
    xs = sx_ref[0]
    ys = sy_ref[0]
    zs = sz_ref[0]

    p2 = Xr * Xr + Yr * Yr + Zr * Zr
    s2 = xs * xs + ys * ys + zs * zs
    cross = xs * Xr + ys * Yr + zs * Zr
    d = s2 + p2 - 2.0 * cross
    msk = d <= r2
    mf = msk.astype(jnp.float32)
    c = _dotf(mf, L_ref[...])
    R = St * K
    c3 = jnp.broadcast_to(c[:, None, :], (St, K, Np))
    m3 = jnp.broadcast_to(msk[:, None, :], (St, K, Np))
    cnt3 = jnp.max(c3, axis=2, keepdims=True)
    kio3 = jax.lax.broadcasted_iota(jnp.int32, (St, K, 1), 1).astype(jnp.float32)
    tau3 = jnp.where(kio3 < cnt3, kio3 + 1.0, 1.0)
    Sel = jnp.where(m3 & (c3 == tau3), 1.0, 0.0).reshape(R, Np)

    G = _dotf(Sel, tab_ref[0])

    lane = jax.lax.broadcasted_iota(jnp.int32, (St, Ct), 1)
    cen = (xs * (lane == 0) + ys * (lane == 1) + zs * (lane == 2))
    cpad = jnp.broadcast_to(cen[:, None, :], (St, K, Ct)).reshape(R, Ct)
    h = G - cpad

    bnscale = jnp.float32(math.sqrt(1.0 + EPS))
    for W, b, g, e in ((W1_ref, b1_ref, g1_ref, e1_ref),
                       (W2_ref, b2_ref, g2_ref, e2_ref),
                       (W3_ref, b3_ref, g3_ref, e3_ref)):
        h = _dotf(h, W[...]) + b[...]
        h = g[...] * h / bnscale + e[...]
        h = jnp.maximum(h, 0.0)

    Cout = out_ref.shape[2]
    out_ref[0] = jnp.max(h.reshape(St, K, Cout), axis=1)


def _ball_sa(tab, Xr, Yr, Zr, sx, sy, sz, L, layers, *, r2, K, St):
    B, Np, Ct = tab.shape
    S = sx.shape[1]
    (W1, b1, g1, e1), (W2, b2, g2, e2), (W3, b3, g3, e3) = layers
    Cout = W3.shape[1]
    vec = lambda v: v.reshape(1, -1)
    grid = (B, S // St)
    c0 = lambda shape: pl.BlockSpec(shape, lambda b, s: (0, 0))
    out = pl.pallas_call(
        functools.partial(_ball_sa_body, Np=Np, K=K, St=St, r2=r2, Ct=Ct),
        grid=grid,
        in_specs=[
            pl.BlockSpec((1, Np, Ct), lambda b, s: (b, 0, 0)),
            pl.BlockSpec((1, 1, Np), lambda b, s: (b, 0, 0)),
            pl.BlockSpec((1, 1, Np), lambda b, s: (b, 0, 0)),
            pl.BlockSpec((1, 1, Np), lambda b, s: (b, 0, 0)),
            pl.BlockSpec((1, St, 1), lambda b, s: (b, s, 0)),
            pl.BlockSpec((1, St, 1), lambda b, s: (b, s, 0)),
            pl.BlockSpec((1, St, 1), lambda b, s: (b, s, 0)),
            c0(L.shape),
            c0(W1.shape), c0((1, W1.shape[1])), c0((1, W1.shape[1])),
            c0((1, W1.shape[1])),
            c0(W2.shape), c0((1, W2.shape[1])), c0((1, W2.shape[1])),
            c0((1, W2.shape[1])),
            c0(W3.shape), c0((1, W3.shape[1])), c0((1, W3.shape[1])),
            c0((1, W3.shape[1])),
        ],
        out_specs=pl.BlockSpec((1, St, Cout), lambda b, s: (b, s, 0)),
        out_shape=jax.ShapeDtypeStruct((B, S, Cout), jnp.float32),
    )(tab,
      Xr.reshape(B, 1, Np), Yr.reshape(B, 1, Np), Zr.reshape(B, 1, Np),
      sx.reshape(B, S, 1), sy.reshape(B, S, 1), sz.reshape(B, S, 1),
      L,
      W1, vec(b1), vec(g1), vec(e1),
      W2, vec(b2), vec(g2), vec(e2),
      W3, vec(b3), vec(g3), vec(e3))
    return out


def _head_body(t_ref, W1_ref, b1_ref, g1_ref, e1_ref, W2_ref, b2_ref, g2_ref,
               e2_ref, W3_ref, b3_ref, g3_ref, e3_ref, f1w_ref, f1b_ref,
               n1g_ref, n1b_ref, f2w_ref, f2b_ref, n2g_ref, n2b_ref, f3w_ref,
               f3b_ref, feat_ref, logp_ref, *, B, Npt):
    h = t_ref[...]
    bnscale = jnp.float32(math.sqrt(1.0 + EPS))
    for W, b, g, e in ((W1_ref, b1_ref, g1_ref, e1_ref),
                       (W2_ref, b2_ref, g2_ref, e2_ref),
                       (W3_ref, b3_ref, g3_ref, e3_ref)):
        h = _dotf(h, W[...]) + b[...]
        h = g[...] * h / bnscale + e[...]
        h = jnp.maximum(h, 0.0)
    C = h.shape[1]
    feat = jnp.max(h.reshape(B, Npt, C), axis=1)
    feat_ref[...] = feat

    a = _dotf(feat, f1w_ref[...]) + f1b_ref[...]
    a = n1g_ref[...] * a / bnscale + n1b_ref[...]
    a = jnp.maximum(a, 0.0)
    a = _dotf(a, f2w_ref[...]) + f2b_ref[...]
    a = n2g_ref[...] * a / bnscale + n2b_ref[...]
    a = jnp.maximum(a, 0.0)
    logits = _dotf(a, f3w_ref[...]) + f3b_ref[...]
    mx = jnp.max(logits, axis=1, keepdims=True)
    sh = logits - jax.lax.stop_gradient(mx)
    logp = sh - jnp.log(jnp.sum(jnp.exp(sh), axis=1, keepdims=True))
    logp_ref[...] = logp


def _head(tab, sa3, fc1, bn1, fc2, bn2, fc3, B):
    Npt = tab.shape[0] // B
    (W1, b1, g1, e1), (W2, b2, g2, e2), (W3, b3, g3, e3) = sa3
    vec = lambda v: v.reshape(1, -1)
    nclass = fc3[0].shape[1]
    return pl.pallas_call(
        functools.partial(_head_body, B=B, Npt=Npt),
        out_shape=(jax.ShapeDtypeStruct((B, W3.shape[1]), jnp.float32),
                   jax.ShapeDtypeStruct((B, nclass), jnp.float32)),
    )(tab,
      W1, vec(b1), vec(g1), vec(e1),
      W2, vec(b2), vec(g2), vec(e2),
      W3, vec(b3), vec(g3), vec(e3),
      fc1[0], vec(fc1[1]), vec(bn1[0]), vec(bn1[1]),
      fc2[0], vec(fc2[1]), vec(bn2[0]), vec(bn2[1]),
      fc3[0], vec(fc3[1]))


def _tri(n):
    r = jnp.arange(n, dtype=jnp.int32)
    return (r[:, None] <= r[None, :]).astype(jnp.float32)


def kernel(xyz, params):
    b, t, k, n = xyz.shape
    B = b * t
    pts = xyz.reshape(B, k, n).transpose(0, 2, 1)
    X = pts[:, :, 0]
    Y = pts[:, :, 1]
    Z = pts[:, :, 2]

    n1x, n1y, n1z = _fps(X, Y, Z, 512)
    L1 = _tri(n)
    l1_pts = _ball_sa(pts, X, Y, Z, n1x, n1y, n1z, L1, params['sa1'],
                      r2=0.2 ** 2, K=32, St=64)

    n2x, n2y, n2z = _fps(n1x, n1y, n1z, 128)
    new1 = jnp.stack([n1x, n1y, n1z], axis=-1)
    tab2 = jnp.concatenate([new1, l1_pts], axis=-1)
    L2 = _tri(512)
    l2_pts = _ball_sa(tab2, n1x, n1y, n1z, n2x, n2y, n2z, L2, params['sa2'],
                      r2=0.4 ** 2, K=64, St=64)

    new2 = jnp.stack([n2x, n2y, n2z], axis=-1)
    tab3 = jnp.concatenate([new2, l2_pts], axis=-1).reshape(B * 128, 259)
    feat, logp = _head(tab3, params['sa3'], params['fc1'], params['bn1'],
                       params['fc2'], params['bn2'], params['fc3'], B)

    pred = logp.reshape(b, t, -1).transpose(0, 2, 1)
    features = feat.reshape(b, t, 1024)
    return pred, features


# speedup vs baseline: 7.1998x; 1.5340x over previous
import functools
import math

import jax
import jax.numpy as jnp
from jax.experimental import pallas as pl

HIGH = jax.lax.Precision.HIGHEST
EPS = 1e-5


def _dotf(a, b):
    return jnp.dot(a, b, precision=HIGH, preferred_element_type=jnp.float32)


def _dot16(a, b):
    return jnp.dot(a, b, precision=jax.lax.Precision.DEFAULT,
                   preferred_element_type=jnp.float32)


def _fps_body(x_ref, y_ref, z_ref, ox_ref, oy_ref, oz_ref, *, S, N, B):
    X = x_ref[...]
    Y = y_ref[...]
    Z = z_ref[...]
    lid = jax.lax.broadcasted_iota(jnp.int32, (B, N), 1)
    sid = jax.lax.broadcasted_iota(jnp.int32, (B, S), 1)

    def body(i, st):
        dist, oh, aX, aY, aZ = st
        cx = jnp.sum(X * oh, axis=1, keepdims=True)
        cy = jnp.sum(Y * oh, axis=1, keepdims=True)
        cz = jnp.sum(Z * oh, axis=1, keepdims=True)
        aX = jnp.where(sid == i, cx, aX)
        aY = jnp.where(sid == i, cy, aY)
        aZ = jnp.where(sid == i, cz, aZ)
        d = (X - cx) ** 2 + (Y - cy) ** 2 + (Z - cz) ** 2
        dist = jnp.minimum(dist, d)
        m = jnp.max(dist, axis=1, keepdims=True)
        cand = jnp.where(dist == m, lid, N)
        idx = jnp.min(cand, axis=1, keepdims=True)
        oh = (lid == idx).astype(jnp.float32)
        return dist, oh, aX, aY, aZ

    dist0 = jnp.full((B, N), 1e10, jnp.float32)
    oh0 = (lid == 0).astype(jnp.float32)
    z = jnp.zeros((B, S), jnp.float32)
    _, _, aX, aY, aZ = jax.lax.fori_loop(0, S, body, (dist0, oh0, z, z, z))
    ox_ref[...] = aX
    oy_ref[...] = aY
    oz_ref[...] = aZ


def _fps(X, Y, Z, S):
    B, N = X.shape
    out = jax.ShapeDtypeStruct((B, S), jnp.float32)
    return pl.pallas_call(
        functools.partial(_fps_body, S=S, N=N, B=B),
        out_shape=(out, out, out),
    )(X, Y, Z)


def _ball_sa_body(tabh_ref, tabl_ref, xr_ref, yr_ref, zr_ref, sx_ref, sy_ref,
                  sz_ref, L_ref, W1_ref, b1_ref, g1_ref, e1_ref, W2_ref,
                  b2_ref, g2_ref, e2_ref, W3_ref, b3_ref, g3_ref, e3_ref,
                  out_ref, *, Np, K, St, r2, Ct):
    Xr = xr_ref[0]
    Yr = yr_ref[0]
    Zr = zr_ref[0]
    xs = sx_ref[0]
    ys = sy_ref[0]
    zs = sz_ref[0]

    p2 = Xr * Xr + Yr * Yr + Zr * Zr
    s2 = xs * xs + ys * ys + zs * zs
    cross = xs * Xr + ys * Yr + zs * Zr
    d = s2 + p2 - 2.0 * cross
    msk = d <= r2
    mf = msk.astype(jnp.float32).astype(jnp.bfloat16)
    c = _dot16(mf, L_ref[...])
    R = St * K
    c3 = jnp.broadcast_to(c[:, None, :], (St, K, Np))
    m3 = jnp.broadcast_to(msk[:, None, :], (St, K, Np))
    cnt3 = jnp.max(c3, axis=2, keepdims=True)
    kio3 = jax.lax.broadcasted_iota(jnp.int32, (St, K, 1), 1).astype(jnp.float32)
    tau3 = jnp.where(kio3 < cnt3, kio3 + 1.0, 1.0)
    Sel = jnp.where(m3 & (c3 == tau3), 1.0, 0.0)
    Sel = Sel.astype(jnp.bfloat16).reshape(R, Np)

    G = _dot16(Sel, tabh_ref[0]) + _dot16(Sel, tabl_ref[0])

    lane = jax.lax.broadcasted_iota(jnp.int32, (St, Ct), 1)
    cen = (xs * (lane == 0) + ys * (lane == 1) + zs * (lane == 2))
    cpad = jnp.broadcast_to(cen[:, None, :], (St, K, Ct)).reshape(R, Ct)
    h = G - cpad

    bnscale = jnp.float32(math.sqrt(1.0 + EPS))
    for W, b, g, e in ((W1_ref, b1_ref, g1_ref, e1_ref),
                       (W2_ref, b2_ref, g2_ref, e2_ref),
                       (W3_ref, b3_ref, g3_ref, e3_ref)):
        h = _dotf(h, W[...]) + b[...]
        h = g[...] * h / bnscale + e[...]
        h = jnp.maximum(h, 0.0)

    Cout = out_ref.shape[2]
    out_ref[0] = jnp.max(h.reshape(St, K, Cout), axis=1)


def _ball_sa(tab, Xr, Yr, Zr, sx, sy, sz, L, layers, *, r2, K, St):
    B, Np, Ct = tab.shape
    S = sx.shape[1]
    (W1, b1, g1, e1), (W2, b2, g2, e2), (W3, b3, g3, e3) = layers
    Cout = W3.shape[1]
    vec = lambda v: v.reshape(1, -1)
    grid = (B, S // St)
    c0 = lambda shape: pl.BlockSpec(shape, lambda b, s: (0, 0))
    tab_hi = tab.astype(jnp.bfloat16)
    tab_lo = (tab - tab_hi.astype(jnp.float32)).astype(jnp.bfloat16)
    out = pl.pallas_call(
        functools.partial(_ball_sa_body, Np=Np, K=K, St=St, r2=r2, Ct=Ct),
        grid=grid,
        in_specs=[
            pl.BlockSpec((1, Np, Ct), lambda b, s: (b, 0, 0)),
            pl.BlockSpec((1, Np, Ct), lambda b, s: (b, 0, 0)),
            pl.BlockSpec((1, 1, Np), lambda b, s: (b, 0, 0)),
            pl.BlockSpec((1, 1, Np), lambda b, s: (b, 0, 0)),
            pl.BlockSpec((1, 1, Np), lambda b, s: (b, 0, 0)),
            pl.BlockSpec((1, St, 1), lambda b, s: (b, s, 0)),
            pl.BlockSpec((1, St, 1), lambda b, s: (b, s, 0)),
            pl.BlockSpec((1, St, 1), lambda b, s: (b, s, 0)),
            c0(L.shape),
            c0(W1.shape), c0((1, W1.shape[1])), c0((1, W1.shape[1])),
            c0((1, W1.shape[1])),
            c0(W2.shape), c0((1, W2.shape[1])), c0((1, W2.shape[1])),
            c0((1, W2.shape[1])),
            c0(W3.shape), c0((1, W3.shape[1])), c0((1, W3.shape[1])),
            c0((1, W3.shape[1])),
        ],
        out_specs=pl.BlockSpec((1, St, Cout), lambda b, s: (b, s, 0)),
        out_shape=jax.ShapeDtypeStruct((B, S, Cout), jnp.float32),
    )(tab_hi, tab_lo,
      Xr.reshape(B, 1, Np), Yr.reshape(B, 1, Np), Zr.reshape(B, 1, Np),
      sx.reshape(B, S, 1), sy.reshape(B, S, 1), sz.reshape(B, S, 1),
      L.astype(jnp.bfloat16),
      W1, vec(b1), vec(g1), vec(e1),
      W2, vec(b2), vec(g2), vec(e2),
      W3, vec(b3), vec(g3), vec(e3))
    return out


def _head_body(t_ref, W1_ref, b1_ref, g1_ref, e1_ref, W2_ref, b2_ref, g2_ref,
               e2_ref, W3_ref, b3_ref, g3_ref, e3_ref, f1w_ref, f1b_ref,
               n1g_ref, n1b_ref, f2w_ref, f2b_ref, n2g_ref, n2b_ref, f3w_ref,
               f3b_ref, feat_ref, logp_ref, *, B, Npt):
    h = t_ref[...]
    bnscale = jnp.float32(math.sqrt(1.0 + EPS))
    for W, b, g, e in ((W1_ref, b1_ref, g1_ref, e1_ref),
                       (W2_ref, b2_ref, g2_ref, e2_ref),
                       (W3_ref, b3_ref, g3_ref, e3_ref)):
        h = _dotf(h, W[...]) + b[...]
        h = g[...] * h / bnscale + e[...]
        h = jnp.maximum(h, 0.0)
    C = h.shape[1]
    feat = jnp.max(h.reshape(B, Npt, C), axis=1)
    feat_ref[...] = feat

    a = _dotf(feat, f1w_ref[...]) + f1b_ref[...]
    a = n1g_ref[...] * a / bnscale + n1b_ref[...]
    a = jnp.maximum(a, 0.0)
    a = _dotf(a, f2w_ref[...]) + f2b_ref[...]
    a = n2g_ref[...] * a / bnscale + n2b_ref[...]
    a = jnp.maximum(a, 0.0)
    logits = _dotf(a, f3w_ref[...]) + f3b_ref[...]
    mx = jnp.max(logits, axis=1, keepdims=True)
    sh = logits - jax.lax.stop_gradient(mx)
    logp = sh - jnp.log(jnp.sum(jnp.exp(sh), axis=1, keepdims=True))
    logp_ref[...] = logp


def _head(tab, sa3, fc1, bn1, fc2, bn2, fc3, B):
    Npt = tab.shape[0] // B
    (W1, b1, g1, e1), (W2, b2, g2, e2), (W3, b3, g3, e3) = sa3
    vec = lambda v: v.reshape(1, -1)
    nclass = fc3[0].shape[1]
    return pl.pallas_call(
        functools.partial(_head_body, B=B, Npt=Npt),
        out_shape=(jax.ShapeDtypeStruct((B, W3.shape[1]), jnp.float32),
                   jax.ShapeDtypeStruct((B, nclass), jnp.float32)),
    )(tab,
      W1, vec(b1), vec(g1), vec(e1),
      W2, vec(b2), vec(g2), vec(e2),
      W3, vec(b3), vec(g3), vec(e3),
      fc1[0], vec(fc1[1]), vec(bn1[0]), vec(bn1[1]),
      fc2[0], vec(fc2[1]), vec(bn2[0]), vec(bn2[1]),
      fc3[0], vec(fc3[1]))


def _tri(n):
    r = jnp.arange(n, dtype=jnp.int32)
    return (r[:, None] <= r[None, :]).astype(jnp.float32)


def kernel(xyz, params):
    b, t, k, n = xyz.shape
    B = b * t
    pts = xyz.reshape(B, k, n).transpose(0, 2, 1)
    X = pts[:, :, 0]
    Y = pts[:, :, 1]
    Z = pts[:, :, 2]

    n1x, n1y, n1z = _fps(X, Y, Z, 512)
    L1 = _tri(n)
    l1_pts = _ball_sa(pts, X, Y, Z, n1x, n1y, n1z, L1, params['sa1'],
                      r2=0.2 ** 2, K=32, St=64)

    n2x, n2y, n2z = _fps(n1x, n1y, n1z, 128)
    new1 = jnp.stack([n1x, n1y, n1z], axis=-1)
    tab2 = jnp.concatenate([new1, l1_pts], axis=-1)
    L2 = _tri(512)
    l2_pts = _ball_sa(tab2, n1x, n1y, n1z, n2x, n2y, n2z, L2, params['sa2'],
                      r2=0.4 ** 2, K=64, St=64)

    new2 = jnp.stack([n2x, n2y, n2z], axis=-1)
    tab3 = jnp.concatenate([new2, l2_pts], axis=-1).reshape(B * 128, 259)
    feat, logp = _head(tab3, params['sa3'], params['fc1'], params['bn1'],
                       params['fc2'], params['bn2'], params['fc3'], B)

    pred = logp.reshape(b, t, -1).transpose(0, 2, 1)
    features = feat.reshape(b, t, 1024)
    return pred, features


# speedup vs baseline: 9.5317x; 1.3239x over previous
import functools
import math

import jax
import jax.numpy as jnp
from jax.experimental import pallas as pl

HIGH = jax.lax.Precision.HIGHEST
EPS = 1e-5


def _dotf(a, b):
    return jnp.dot(a, b, precision=HIGH, preferred_element_type=jnp.float32)


def _dot16(a, b):
    return jnp.dot(a, b, precision=jax.lax.Precision.DEFAULT,
                   preferred_element_type=jnp.float32)


def _dot3(a, b):
    ah = a.astype(jnp.bfloat16)
    al = (a - ah.astype(jnp.float32)).astype(jnp.bfloat16)
    bh = b.astype(jnp.bfloat16)
    bl = (b - bh.astype(jnp.float32)).astype(jnp.bfloat16)
    return _dot16(ah, bh) + (_dot16(ah, bl) + _dot16(al, bh))


def _fps_body(x_ref, y_ref, z_ref, ox_ref, oy_ref, oz_ref, *, S, N, B):
    X = x_ref[...]
    Y = y_ref[...]
    Z = z_ref[...]
    lid = jax.lax.broadcasted_iota(jnp.int32, (B, N), 1)
    sid = jax.lax.broadcasted_iota(jnp.int32, (B, S), 1)

    def body(i, st):
        dist, oh, aX, aY, aZ = st
        cx = jnp.sum(X * oh, axis=1, keepdims=True)
        cy = jnp.sum(Y * oh, axis=1, keepdims=True)
        cz = jnp.sum(Z * oh, axis=1, keepdims=True)
        aX = jnp.where(sid == i, cx, aX)
        aY = jnp.where(sid == i, cy, aY)
        aZ = jnp.where(sid == i, cz, aZ)
        d = (X - cx) ** 2 + (Y - cy) ** 2 + (Z - cz) ** 2
        dist = jnp.minimum(dist, d)
        m = jnp.max(dist, axis=1, keepdims=True)
        cand = jnp.where(dist == m, lid, N)
        idx = jnp.min(cand, axis=1, keepdims=True)
        oh = (lid == idx).astype(jnp.float32)
        return dist, oh, aX, aY, aZ

    dist0 = jnp.full((B, N), 1e10, jnp.float32)
    oh0 = (lid == 0).astype(jnp.float32)
    z = jnp.zeros((B, S), jnp.float32)
    _, _, aX, aY, aZ = jax.lax.fori_loop(0, S, body, (dist0, oh0, z, z, z))
    ox_ref[...] = aX
    oy_ref[...] = aY
    oz_ref[...] = aZ


def _fps(X, Y, Z, S):
    B, N = X.shape
    out = jax.ShapeDtypeStruct((B, S), jnp.float32)
    return pl.pallas_call(
        functools.partial(_fps_body, S=S, N=N, B=B),
        out_shape=(out, out, out),
    )(X, Y, Z)


def _ball_sa_body(tabh_ref, tabl_ref, xr_ref, yr_ref, zr_ref, sx_ref, sy_ref,
                  sz_ref, L_ref, W1_ref, b1_ref, g1_ref, e1_ref, W2_ref,
                  b2_ref, g2_ref, e2_ref, W3_ref, b3_ref, g3_ref, e3_ref,
                  out_ref, *, Np, K, St, r2, Ct):
    Xr = xr_ref[0]
    Yr = yr_ref[0]
    Zr = zr_ref[0]
    xs = sx_ref[0]
    ys = sy_ref[0]
    zs = sz_ref[0]

    p2 = Xr * Xr + Yr * Yr + Zr * Zr
    s2 = xs * xs + ys * ys + zs * zs
    cross = xs * Xr + ys * Yr + zs * Zr
    d = s2 + p2 - 2.0 * cross
    msk = d <= r2
    mf = msk.astype(jnp.float32).astype(jnp.bfloat16)
    c = _dot16(mf, L_ref[...])
    R = St * K
    cnt3 = jnp.max(c, axis=1, keepdims=True)[:, None, :]
    kio3 = jax.lax.broadcasted_iota(jnp.int32, (St, K, 1), 1).astype(jnp.float32)
    tau3 = jnp.where(kio3 < cnt3, kio3 + 1.0, 1.0)
    c3 = jnp.broadcast_to(c[:, None, :], (St, K, Np))
    m3 = jnp.broadcast_to(msk[:, None, :], (St, K, Np))
    Sel = (m3 & (c3 == tau3)).astype(jnp.bfloat16).reshape(R, Np)

    G = _dot16(Sel, tabh_ref[0]) + _dot16(Sel, tabl_ref[0])

    lane = jax.lax.broadcasted_iota(jnp.int32, (St, Ct), 1)
    cen = (xs * (lane == 0) + ys * (lane == 1) + zs * (lane == 2))
    cpad = jnp.broadcast_to(cen[:, None, :], (St, K, Ct)).reshape(R, Ct)
    h = G - cpad

    bnscale = jnp.float32(math.sqrt(1.0 + EPS))
    for W, b, g, e in ((W1_ref, b1_ref, g1_ref, e1_ref),
                       (W2_ref, b2_ref, g2_ref, e2_ref),
                       (W3_ref, b3_ref, g3_ref, e3_ref)):
        h = _dot3(h, W[...]) + b[...]
        h = g[...] * h / bnscale + e[...]
        h = jnp.maximum(h, 0.0)

    Cout = out_ref.shape[2]
    out_ref[0] = jnp.max(h.reshape(St, K, Cout), axis=1)


def _ball_sa(tab, Xr, Yr, Zr, sx, sy, sz, L, layers, *, r2, K, St):
    B, Np, Ct = tab.shape
    S = sx.shape[1]
    (W1, b1, g1, e1), (W2, b2, g2, e2), (W3, b3, g3, e3) = layers
    Cout = W3.shape[1]
    vec = lambda v: v.reshape(1, -1)
    grid = (B, S // St)
    c0 = lambda shape: pl.BlockSpec(shape, lambda b, s: (0, 0))
    tab_hi = tab.astype(jnp.bfloat16)
    tab_lo = (tab - tab_hi.astype(jnp.float32)).astype(jnp.bfloat16)
    out = pl.pallas_call(
        functools.partial(_ball_sa_body, Np=Np, K=K, St=St, r2=r2, Ct=Ct),
        grid=grid,
        in_specs=[
            pl.BlockSpec((1, Np, Ct), lambda b, s: (b, 0, 0)),
            pl.BlockSpec((1, Np, Ct), lambda b, s: (b, 0, 0)),
            pl.BlockSpec((1, 1, Np), lambda b, s: (b, 0, 0)),
            pl.BlockSpec((1, 1, Np), lambda b, s: (b, 0, 0)),
            pl.BlockSpec((1, 1, Np), lambda b, s: (b, 0, 0)),
            pl.BlockSpec((1, St, 1), lambda b, s: (b, s, 0)),
            pl.BlockSpec((1, St, 1), lambda b, s: (b, s, 0)),
            pl.BlockSpec((1, St, 1), lambda b, s: (b, s, 0)),
            c0(L.shape),
            c0(W1.shape), c0((1, W1.shape[1])), c0((1, W1.shape[1])),
            c0((1, W1.shape[1])),
            c0(W2.shape), c0((1, W2.shape[1])), c0((1, W2.shape[1])),
            c0((1, W2.shape[1])),
            c0(W3.shape), c0((1, W3.shape[1])), c0((1, W3.shape[1])),
            c0((1, W3.shape[1])),
        ],
        out_specs=pl.BlockSpec((1, St, Cout), lambda b, s: (b, s, 0)),
        out_shape=jax.ShapeDtypeStruct((B, S, Cout), jnp.float32),
    )(tab_hi, tab_lo,
      Xr.reshape(B, 1, Np), Yr.reshape(B, 1, Np), Zr.reshape(B, 1, Np),
      sx.reshape(B, S, 1), sy.reshape(B, S, 1), sz.reshape(B, S, 1),
      L.astype(jnp.bfloat16),
      W1, vec(b1), vec(g1), vec(e1),
      W2, vec(b2), vec(g2), vec(e2),
      W3, vec(b3), vec(g3), vec(e3))
    return out


def _head_body(t_ref, W1_ref, b1_ref, g1_ref, e1_ref, W2_ref, b2_ref, g2_ref,
               e2_ref, W3_ref, b3_ref, g3_ref, e3_ref, f1w_ref, f1b_ref,
               n1g_ref, n1b_ref, f2w_ref, f2b_ref, n2g_ref, n2b_ref, f3w_ref,
               f3b_ref, feat_ref, logp_ref, *, B, Npt):
    h = t_ref[...]
    bnscale = jnp.float32(math.sqrt(1.0 + EPS))
    for W, b, g, e in ((W1_ref, b1_ref, g1_ref, e1_ref),
                       (W2_ref, b2_ref, g2_ref, e2_ref),
                       (W3_ref, b3_ref, g3_ref, e3_ref)):
        h = _dot3(h, W[...]) + b[...]
        h = g[...] * h / bnscale + e[...]
        h = jnp.maximum(h, 0.0)
    C = h.shape[1]
    feat = jnp.max(h.reshape(B, Npt, C), axis=1)
    feat_ref[...] = feat

    a = _dot3(feat, f1w_ref[...]) + f1b_ref[...]
    a = n1g_ref[...] * a / bnscale + n1b_ref[...]
    a = jnp.maximum(a, 0.0)
    a = _dot3(a, f2w_ref[...]) + f2b_ref[...]
    a = n2g_ref[...] * a / bnscale + n2b_ref[...]
    a = jnp.maximum(a, 0.0)
    logits = _dot3(a, f3w_ref[...]) + f3b_ref[...]
    mx = jnp.max(logits, axis=1, keepdims=True)
    sh = logits - jax.lax.stop_gradient(mx)
    logp = sh - jnp.log(jnp.sum(jnp.exp(sh), axis=1, keepdims=True))
    logp_ref[...] = logp


def _head(tab, sa3, fc1, bn1, fc2, bn2, fc3, B):
    Npt = tab.shape[0] // B
    (W1, b1, g1, e1), (W2, b2, g2, e2), (W3, b3, g3, e3) = sa3
    vec = lambda v: v.reshape(1, -1)
    nclass = fc3[0].shape[1]
    return pl.pallas_call(
        functools.partial(_head_body, B=B, Npt=Npt),
        out_shape=(jax.ShapeDtypeStruct((B, W3.shape[1]), jnp.float32),
                   jax.ShapeDtypeStruct((B, nclass), jnp.float32)),
    )(tab,
      W1, vec(b1), vec(g1), vec(e1),
      W2, vec(b2), vec(g2), vec(e2),
      W3, vec(b3), vec(g3), vec(e3),
      fc1[0], vec(fc1[1]), vec(bn1[0]), vec(bn1[1]),
      fc2[0], vec(fc2[1]), vec(bn2[0]), vec(bn2[1]),
      fc3[0], vec(fc3[1]))


def _tri(n):
    r = jnp.arange(n, dtype=jnp.int32)
    return (r[:, None] <= r[None, :]).astype(jnp.float32)


def kernel(xyz, params):
    b, t, k, n = xyz.shape
    B = b * t
    pts = xyz.reshape(B, k, n).transpose(0, 2, 1)
    X = pts[:, :, 0]
    Y = pts[:, :, 1]
    Z = pts[:, :, 2]

    n1x, n1y, n1z = _fps(X, Y, Z, 512)
    L1 = _tri(n)
    l1_pts = _ball_sa(pts, X, Y, Z, n1x, n1y, n1z, L1, params['sa1'],
                      r2=0.2 ** 2, K=32, St=64)

    n2x, n2y, n2z = _fps(n1x, n1y, n1z, 128)
    new1 = jnp.stack([n1x, n1y, n1z], axis=-1)
    tab2 = jnp.concatenate([new1, l1_pts], axis=-1)
    L2 = _tri(512)
    l2_pts = _ball_sa(tab2, n1x, n1y, n1z, n2x, n2y, n2z, L2, params['sa2'],
                      r2=0.4 ** 2, K=64, St=64)

    new2 = jnp.stack([n2x, n2y, n2z], axis=-1)
    tab3 = jnp.concatenate([new2, l2_pts], axis=-1).reshape(B * 128, 259)
    feat, logp = _head(tab3, params['sa3'], params['fc1'], params['bn1'],
                       params['fc2'], params['bn2'], params['fc3'], B)

    pred = logp.reshape(b, t, -1).transpose(0, 2, 1)
    features = feat.reshape(b, t, 1024)
    return pred, features


# speedup vs baseline: 10.3514x; 1.0860x over previous
import functools
import math

import jax
import jax.numpy as jnp
from jax.experimental import pallas as pl

HIGHEST = jax.lax.Precision.HIGHEST
EPS = 1e-5


def _dotf(a, b):
    return jnp.dot(a, b, precision=HIGHEST, preferred_element_type=jnp.float32)


def _dot16(a, b):
    return jnp.dot(a, b, precision=jax.lax.Precision.DEFAULT,
                   preferred_element_type=jnp.float32)


def _split_hi_lo(w):
    hi = w.astype(jnp.bfloat16)
    lo = (w - hi.astype(jnp.float32)).astype(jnp.bfloat16)
    return hi, lo


def _dot3(a, bh, bl):
    ah = a.astype(jnp.bfloat16)
    al = (a - ah.astype(jnp.float32)).astype(jnp.bfloat16)
    return _dot16(ah, bh) + (_dot16(ah, bl) + _dot16(al, bh))


def _fps_body(xyz3_ref, oi_ref, *, S, N, B):
    XYZ3 = xyz3_ref[...]
    X = XYZ3[0:B]
    Y = XYZ3[B:2 * B]
    Z = XYZ3[2 * B:3 * B]
    lid = jax.lax.broadcasted_iota(jnp.int32, (B, N), 1)
    sid = jax.lax.broadcasted_iota(jnp.int32, (B, S), 1)

    def body(i, st):
        dist, idx, aI = st
        aI = aI + (sid == i).astype(jnp.int32) * idx
        oh = (lid == idx).astype(jnp.float32)
        oh3 = jnp.concatenate([oh, oh, oh], axis=0)
        red = jnp.sum(XYZ3 * oh3, axis=1, keepdims=True)
        cx = red[0:B]
        cy = red[B:2 * B]
        cz = red[2 * B:3 * B]
        d = (X - cx) ** 2 + (Y - cy) ** 2 + (Z - cz) ** 2
        dist = jnp.minimum(dist, d)
        m = jnp.max(dist, axis=1, keepdims=True)
        idx = jnp.min(jnp.where(dist == m, lid, N), axis=1, keepdims=True)
        return dist, idx, aI

    dist0 = jnp.full((B, N), 1e10, jnp.float32)
    bidN = jax.lax.broadcasted_iota(jnp.int32, (B, N), 0)
    bidS = jax.lax.broadcasted_iota(jnp.int32, (B, S), 0)
    idx0 = jnp.min(lid * bidN, axis=1, keepdims=True)
    aI0 = jnp.minimum(sid * bidS, 0)
    _, _, aI = jax.lax.fori_loop(0, S, body, (dist0, idx0, aI0))
    oi_ref[...] = aI


def _fps_idx(X, Y, Z, S):
    B, N = X.shape
    xyz3 = jnp.concatenate([X, Y, Z], axis=0)
    return pl.pallas_call(
        functools.partial(_fps_body, S=S, N=N, B=B),
        out_shape=jax.ShapeDtypeStruct((B, S), jnp.int32),
    )(xyz3)


def _gather_body(idx_ref, pts_ref, out_ref, *, S, Np):
    idxv = idx_ref[0]
    nio = jax.lax.broadcasted_iota(jnp.int32, (S, Np), 1)
    OH = (nio == idxv).astype(jnp.float32)
    out_ref[0] = _dotf(OH, pts_ref[0])


def _gather_rows(idx, pts):
    B, Np, C = pts.shape
    S = idx.shape[1]
    return pl.pallas_call(
        functools.partial(_gather_body, S=S, Np=Np),
        grid=(B,),
        in_specs=[
            pl.BlockSpec((1, S, 1), lambda b: (b, 0, 0)),
            pl.BlockSpec((1, Np, C), lambda b: (b, 0, 0)),
        ],
        out_specs=pl.BlockSpec((1, S, C), lambda b: (b, 0, 0)),
        out_shape=jax.ShapeDtypeStruct((B, S, C), jnp.float32),
    )(idx.reshape(B, S, 1), pts)


def _fps(X, Y, Z, S, pts):
    idx = _fps_idx(X, Y, Z, S)
    new = _gather_rows(idx, pts)
    return new[:, :, 0], new[:, :, 1], new[:, :, 2]


def _ball_sa_body(tab_ref, xr_ref, yr_ref, zr_ref, sx_ref, sy_ref,
                  sz_ref, L_ref, W1h_ref, W1l_ref, b1_ref, g1_ref, e1_ref,
                  W2h_ref, W2l_ref, b2_ref, g2_ref, e2_ref, W3h_ref, W3l_ref,
                  b3_ref, g3_ref, e3_ref, out_ref, *, Np, K, St, r2, Ct, Off):
    Xr = xr_ref[0]
    Yr = yr_ref[0]
    Zr = zr_ref[0]
    xs = sx_ref[0]
    ys = sy_ref[0]
    zs = sz_ref[0]

    p2 = Xr * Xr + Yr * Yr + Zr * Zr
    s2 = xs * xs + ys * ys + zs * zs
    cross = xs * Xr + ys * Yr + zs * Zr
    d = s2 + p2 - 2.0 * cross
    msk = d <= r2
    mf = msk.astype(jnp.float32).astype(jnp.bfloat16)
    c = _dot16(mf, L_ref[...])
    R = St * K
    c16 = c.astype(jnp.bfloat16)
    cnt3 = jnp.max(c, axis=1, keepdims=True)[:, None, :]
    kio3 = jax.lax.broadcasted_iota(jnp.int32, (St, K, 1), 1).astype(jnp.float32)
    tau3 = jnp.where(kio3 < cnt3, kio3 + 1.0, 1.0).astype(jnp.bfloat16)
    c3 = jnp.broadcast_to(c16[:, None, :], (St, K, Np))
    m3 = jnp.broadcast_to(mf[:, None, :], (St, K, Np))
    Sel = ((c3 == tau3) & (m3 > 0)).astype(jnp.bfloat16).reshape(R, Np)

    Ghl = _dot16(Sel, tab_ref[0])
    G = Ghl[:, 0:Ct] + Ghl[:, Off:Off + Ct]

    lane = jax.lax.broadcasted_iota(jnp.int32, (St, Ct), 1)
    cen = (xs * (lane == 0) + ys * (lane == 1) + zs * (lane == 2))
    cpad = jnp.broadcast_to(cen[:, None, :], (St, K, Ct)).reshape(R, Ct)
    h = G - cpad

    bnscale = jnp.float32(math.sqrt(1.0 + EPS))
    for Wh, Wl, b, g, e in (
            (W1h_ref, W1l_ref, b1_ref, g1_ref, e1_ref),
            (W2h_ref, W2l_ref, b2_ref, g2_ref, e2_ref),
            (W3h_ref, W3l_ref, b3_ref, g3_ref, e3_ref)):
        h = _dot3(h, Wh[...], Wl[...]) + b[...]
        h = g[...] * h / bnscale + e[...]
        h = jnp.maximum(h, 0.0)

    Cout = out_ref.shape[2]
    out_ref[0] = jnp.max(h.reshape(St, K, Cout), axis=1)


def _ball_sa(tab, Xr, Yr, Zr, sx, sy, sz, L, layers, *, r2, K, St):
    B, Np, Ct = tab.shape
    S = sx.shape[1]
    (W1, b1, g1, e1), (W2, b2, g2, e2), (W3, b3, g3, e3) = layers
    Cout = W3.shape[1]
    vec = lambda v: v.reshape(1, -1)
    grid = (B, S // St)
    c0 = lambda shape: pl.BlockSpec(shape, lambda b, s: (0, 0))

    Off = 128 if Ct <= 128 else 256
    tab_hi = tab.astype(jnp.bfloat16)
    tab_lo = (tab - tab_hi.astype(jnp.float32)).astype(jnp.bfloat16)
    pad = jnp.zeros((B, Np, Off - Ct), jnp.bfloat16)
    tab_hl = jnp.concatenate([tab_hi, pad, tab_lo], axis=-1)

    W1h, W1l = _split_hi_lo(W1)
    W2h, W2l = _split_hi_lo(W2)
    W3h, W3l = _split_hi_lo(W3)

    out = pl.pallas_call(
        functools.partial(_ball_sa_body, Np=Np, K=K, St=St, r2=r2, Ct=Ct,
                          Off=Off),
        grid=grid,
        in_specs=[
            pl.BlockSpec((1, Np, Off + Ct), lambda b, s: (b, 0, 0)),
            pl.BlockSpec((1, 1, Np), lambda b, s: (b, 0, 0)),
            pl.BlockSpec((1, 1, Np), lambda b, s: (b, 0, 0)),
            pl.BlockSpec((1, 1, Np), lambda b, s: (b, 0, 0)),
            pl.BlockSpec((1, St, 1), lambda b, s: (b, s, 0)),
            pl.BlockSpec((1, St, 1), lambda b, s: (b, s, 0)),
            pl.BlockSpec((1, St, 1), lambda b, s: (b, s, 0)),
            c0(L.shape),
            c0(W1.shape), c0(W1.shape), c0((1, W1.shape[1])),
            c0((1, W1.shape[1])), c0((1, W1.shape[1])),
            c0(W2.shape), c0(W2.shape), c0((1, W2.shape[1])),
            c0((1, W2.shape[1])), c0((1, W2.shape[1])),
            c0(W3.shape), c0(W3.shape), c0((1, W3.shape[1])),
            c0((1, W3.shape[1])), c0((1, W3.shape[1])),
        ],
        out_specs=pl.BlockSpec((1, St, Cout), lambda b, s: (b, s, 0)),
        out_shape=jax.ShapeDtypeStruct((B, S, Cout), jnp.float32),
    )(tab_hl,
      Xr.reshape(B, 1, Np), Yr.reshape(B, 1, Np), Zr.reshape(B, 1, Np),
      sx.reshape(B, S, 1), sy.reshape(B, S, 1), sz.reshape(B, S, 1),
      L.astype(jnp.bfloat16),
      W1h, W1l, vec(b1), vec(g1), vec(e1),
      W2h, W2l, vec(b2), vec(g2), vec(e2),
      W3h, W3l, vec(b3), vec(g3), vec(e3))
    return out


def _head_body(t_ref, W1h_ref, W1l_ref, b1_ref, g1_ref, e1_ref, W2h_ref,
               W2l_ref, b2_ref, g2_ref, e2_ref, W3h_ref, W3l_ref, b3_ref,
               g3_ref, e3_ref, f1h_ref, f1l_ref, f1b_ref, n1g_ref, n1b_ref,
               f2h_ref, f2l_ref, f2b_ref, n2g_ref, n2b_ref, f3h_ref, f3l_ref,
               f3b_ref, feat_ref, logp_ref, *, B, Npt):
    h = t_ref[...]
    bnscale = jnp.float32(math.sqrt(1.0 + EPS))
    for Wh, Wl, b, g, e in (
            (W1h_ref, W1l_ref, b1_ref, g1_ref, e1_ref),
            (W2h_ref, W2l_ref, b2_ref, g2_ref, e2_ref),
            (W3h_ref, W3l_ref, b3_ref, g3_ref, e3_ref)):
        h = _dot3(h, Wh[...], Wl[...]) + b[...]
        h = g[...] * h / bnscale + e[...]
        h = jnp.maximum(h, 0.0)
    C = h.shape[1]
    feat = jnp.max(h.reshape(B, Npt, C), axis=1)
    feat_ref[...] = feat

    a = _dot3(feat, f1h_ref[...], f1l_ref[...]) + f1b_ref[...]
    a = n1g_ref[...] * a / bnscale + n1b_ref[...]
    a = jnp.maximum(a, 0.0)
    a = _dot3(a, f2h_ref[...], f2l_ref[...]) + f2b_ref[...]
    a = n2g_ref[...] * a / bnscale + n2b_ref[...]
    a = jnp.maximum(a, 0.0)
    logits = _dot3(a, f3h_ref[...], f3l_ref[...]) + f3b_ref[...]
    mx = jnp.max(logits, axis=1, keepdims=True)
    sh = logits - jax.lax.stop_gradient(mx)
    logp = sh - jnp.log(jnp.sum(jnp.exp(sh), axis=1, keepdims=True))
    logp_ref[...] = logp


def _head(tab, sa3, fc1, bn1, fc2, bn2, fc3, B):
    Npt = tab.shape[0] // B
    (W1, b1, g1, e1), (W2, b2, g2, e2), (W3, b3, g3, e3) = sa3
    vec = lambda v: v.reshape(1, -1)
    nclass = fc3[0].shape[1]
    W1h, W1l = _split_hi_lo(W1)
    W2h, W2l = _split_hi_lo(W2)
    W3h, W3l = _split_hi_lo(W3)
    f1h, f1l = _split_hi_lo(fc1[0])
    f2h, f2l = _split_hi_lo(fc2[0])
    f3h, f3l = _split_hi_lo(fc3[0])
    return pl.pallas_call(
        functools.partial(_head_body, B=B, Npt=Npt),
        out_shape=(jax.ShapeDtypeStruct((B, W3.shape[1]), jnp.float32),
                   jax.ShapeDtypeStruct((B, nclass), jnp.float32)),
    )(tab,
      W1h, W1l, vec(b1), vec(g1), vec(e1),
      W2h, W2l, vec(b2), vec(g2), vec(e2),
      W3h, W3l, vec(b3), vec(g3), vec(e3),
      f1h, f1l, vec(fc1[1]), vec(bn1[0]), vec(bn1[1]),
      f2h, f2l, vec(fc2[1]), vec(bn2[0]), vec(bn2[1]),
      f3h, f3l, vec(fc3[1]))


def _tri(n):
    r = jnp.arange(n, dtype=jnp.int32)
    return (r[:, None] <= r[None, :]).astype(jnp.float32)


def kernel(xyz, params):
    b, t, k, n = xyz.shape
    B = b * t
    pts = xyz.reshape(B, k, n).transpose(0, 2, 1)
    X = pts[:, :, 0]
    Y = pts[:, :, 1]
    Z = pts[:, :, 2]

    n1x, n1y, n1z = _fps(X, Y, Z, 512, pts)
    L1 = _tri(n)
    l1_pts = _ball_sa(pts, X, Y, Z, n1x, n1y, n1z, L1, params['sa1'],
                      r2=0.2 ** 2, K=32, St=64)

    new1 = jnp.stack([n1x, n1y, n1z], axis=-1)
    n2x, n2y, n2z = _fps(n1x, n1y, n1z, 128, new1)
    tab2 = jnp.concatenate([new1, l1_pts], axis=-1)
    L2 = _tri(512)
    l2_pts = _ball_sa(tab2, n1x, n1y, n1z, n2x, n2y, n2z, L2, params['sa2'],
                      r2=0.4 ** 2, K=64, St=64)

    new2 = jnp.stack([n2x, n2y, n2z], axis=-1)
    tab3 = jnp.concatenate([new2, l2_pts], axis=-1).reshape(B * 128, 259)
    feat, logp = _head(tab3, params['sa3'], params['fc1'], params['bn1'],
                       params['fc2'], params['bn2'], params['fc3'], B)

    pred = logp.reshape(b, t, -1).transpose(0, 2, 1)
    features = feat.reshape(b, t, 1024)
    return pred, features


# speedup vs baseline: 10.7754x; 1.0410x over previous
import functools
import math

import jax
import jax.numpy as jnp
from jax import lax
from jax.experimental import pallas as pl
from jax.experimental.pallas import tpu as pltpu
from jax.experimental.pallas import tpu_sc as plsc

HIGHEST = jax.lax.Precision.HIGHEST
EPS = 1e-5


def _dotf(a, b):
    return jnp.dot(a, b, precision=HIGHEST, preferred_element_type=jnp.float32)


def _dot16(a, b):
    return jnp.dot(a, b, precision=jax.lax.Precision.DEFAULT,
                   preferred_element_type=jnp.float32)


def _split_hi_lo(w):
    hi = w.astype(jnp.bfloat16)
    lo = (w - hi.astype(jnp.float32)).astype(jnp.bfloat16)
    return hi, lo


def _dot3(a, bh, bl):
    ah = a.astype(jnp.bfloat16)
    al = (a - ah.astype(jnp.float32)).astype(jnp.bfloat16)
    return _dot16(ah, bh) + (_dot16(ah, bl) + _dot16(al, bh))


def _fps_body(xyz3_ref, oi_ref, *, S, N, B):
    XYZ3 = xyz3_ref[...]
    X = XYZ3[0:B]
    Y = XYZ3[B:2 * B]
    Z = XYZ3[2 * B:3 * B]
    lid = jax.lax.broadcasted_iota(jnp.int32, (B, N), 1)
    sid = jax.lax.broadcasted_iota(jnp.int32, (B, S), 1)

    def body(i, st):
        dist, idx, aI = st
        aI = aI + (sid == i).astype(jnp.int32) * idx
        oh = (lid == idx).astype(jnp.float32)
        oh3 = jnp.concatenate([oh, oh, oh], axis=0)
        red = jnp.sum(XYZ3 * oh3, axis=1, keepdims=True)
        cx = red[0:B]
        cy = red[B:2 * B]
        cz = red[2 * B:3 * B]
        d = (X - cx) ** 2 + (Y - cy) ** 2 + (Z - cz) ** 2
        dist = jnp.minimum(dist, d)
        m = jnp.max(dist, axis=1, keepdims=True)
        idx = jnp.min(jnp.where(dist == m, lid, N), axis=1, keepdims=True)
        return dist, idx, aI

    dist0 = jnp.full((B, N), 1e10, jnp.float32)
    bidN = jax.lax.broadcasted_iota(jnp.int32, (B, N), 0)
    bidS = jax.lax.broadcasted_iota(jnp.int32, (B, S), 0)
    idx0 = jnp.min(lid * bidN, axis=1, keepdims=True)
    aI0 = jnp.minimum(sid * bidS, 0)
    _, _, aI = jax.lax.fori_loop(0, S, body, (dist0, idx0, aI0))
    oi_ref[...] = aI


def _fps_idx(X, Y, Z, S):
    B, N = X.shape
    xyz3 = jnp.concatenate([X, Y, Z], axis=0)
    return pl.pallas_call(
        functools.partial(_fps_body, S=S, N=N, B=B),
        out_shape=jax.ShapeDtypeStruct((B, S), jnp.int32),
    )(xyz3)


def _gather_body(idx_ref, pts_ref, out_ref, *, S, Np):
    idxv = idx_ref[0]
    nio = jax.lax.broadcasted_iota(jnp.int32, (S, Np), 1)
    OH = (nio == idxv).astype(jnp.float32)
    out_ref[0] = _dotf(OH, pts_ref[0])


def _gather_rows(idx, pts):
    B, Np, C = pts.shape
    S = idx.shape[1]
    return pl.pallas_call(
        functools.partial(_gather_body, S=S, Np=Np),
        grid=(B,),
        in_specs=[
            pl.BlockSpec((1, S, 1), lambda b: (b, 0, 0)),
            pl.BlockSpec((1, Np, C), lambda b: (b, 0, 0)),
        ],
        out_specs=pl.BlockSpec((1, S, C), lambda b: (b, 0, 0)),
        out_shape=jax.ShapeDtypeStruct((B, S, C), jnp.float32),
    )(idx.reshape(B, S, 1), pts)


def _sc_gather_rows(idx, pts):
    B, Np, C = pts.shape
    S = idx.shape[1]
    D = 128
    tbl = jnp.concatenate(
        [pts, jnp.zeros((B, Np, D - C), jnp.float32)], axis=-1)
    tbl = tbl.reshape(B * Np, D)
    gidx = (idx + (jnp.arange(B, dtype=jnp.int32) * Np)[:, None]).reshape(-1)
    BT = B * S
    NW = 32
    b_per_w = BT // NW
    mesh = plsc.VectorSubcoreMesh(core_axis_name="c", subcore_axis_name="s")

    @functools.partial(
        pl.kernel, mesh=mesh,
        out_type=jax.ShapeDtypeStruct((BT, D), jnp.float32),
        scratch_types=[
            pltpu.VMEM((b_per_w,), jnp.int32),
            pltpu.VMEM((b_per_w, D), jnp.float32),
            pltpu.SemaphoreType.DMA,
        ],
    )
    def k(table_hbm, idx_hbm, out_hbm, idx_v, rows_v, sem):
        wid = lax.axis_index("s") * 2 + lax.axis_index("c")
        base = wid * b_per_w
        pltpu.sync_copy(idx_hbm.at[pl.ds(base, b_per_w)], idx_v)
        pltpu.async_copy(table_hbm.at[idx_v], rows_v, sem).wait()
        pltpu.sync_copy(rows_v, out_hbm.at[pl.ds(base, b_per_w)])

    return k(tbl, gidx).reshape(B, S, D)


def _fps(X, Y, Z, S, pts):
    idx = _fps_idx(X, Y, Z, S)
    new = _sc_gather_rows(idx, pts)
    return new[:, :, 0], new[:, :, 1], new[:, :, 2]


def _ball_sa_body(tab_ref, xr_ref, yr_ref, zr_ref, sx_ref, sy_ref,
                  sz_ref, L_ref, W1h_ref, W1l_ref, b1_ref, g1_ref, e1_ref,
                  W2h_ref, W2l_ref, b2_ref, g2_ref, e2_ref, W3h_ref, W3l_ref,
                  b3_ref, g3_ref, e3_ref, out_ref, *, Np, K, St, r2, Ct, Off):
    Xr = xr_ref[0]
    Yr = yr_ref[0]
    Zr = zr_ref[0]
    xs = sx_ref[0]
    ys = sy_ref[0]
    zs = sz_ref[0]

    p2 = Xr * Xr + Yr * Yr + Zr * Zr
    s2 = xs * xs + ys * ys + zs * zs
    cross = xs * Xr + ys * Yr + zs * Zr
    d = s2 + p2 - 2.0 * cross
    msk = d <= r2
    mf = msk.astype(jnp.float32).astype(jnp.bfloat16)
    c = _dot16(mf, L_ref[...])
    R = St * K
    c16 = c.astype(jnp.bfloat16)
    cnt3 = jnp.max(c, axis=1, keepdims=True)[:, None, :]
    kio3 = jax.lax.broadcasted_iota(jnp.int32, (St, K, 1), 1).astype(jnp.float32)
    tau3 = jnp.where(kio3 < cnt3, kio3 + 1.0, 1.0).astype(jnp.bfloat16)
    c3 = jnp.broadcast_to(c16[:, None, :], (St, K, Np))
    m3 = jnp.broadcast_to(mf[:, None, :], (St, K, Np))
    Sel = ((c3 == tau3) & (m3 > 0)).astype(jnp.bfloat16).reshape(R, Np)

    Ghl = _dot16(Sel, tab_ref[0])
    G = Ghl[:, 0:Ct] + Ghl[:, Off:Off + Ct]

    lane = jax.lax.broadcasted_iota(jnp.int32, (St, Ct), 1)
    cen = (xs * (lane == 0) + ys * (lane == 1) + zs * (lane == 2))
    cpad = jnp.broadcast_to(cen[:, None, :], (St, K, Ct)).reshape(R, Ct)
    h = G - cpad

    bnscale = jnp.float32(math.sqrt(1.0 + EPS))
    for Wh, Wl, b, g, e in (
            (W1h_ref, W1l_ref, b1_ref, g1_ref, e1_ref),
            (W2h_ref, W2l_ref, b2_ref, g2_ref, e2_ref),
            (W3h_ref, W3l_ref, b3_ref, g3_ref, e3_ref)):
        h = _dot3(h, Wh[...], Wl[...]) + b[...]
        h = g[...] * h / bnscale + e[...]
        h = jnp.maximum(h, 0.0)

    Cout = out_ref.shape[2]
    out_ref[0] = jnp.max(h.reshape(St, K, Cout), axis=1)


def _ball_sa(tab, Xr, Yr, Zr, sx, sy, sz, L, layers, *, r2, K, St):
    B, Np, Ct = tab.shape
    S = sx.shape[1]
    (W1, b1, g1, e1), (W2, b2, g2, e2), (W3, b3, g3, e3) = layers
    Cout = W3.shape[1]
    vec = lambda v: v.reshape(1, -1)
    grid = (B, S // St)
    c0 = lambda shape: pl.BlockSpec(shape, lambda b, s: (0, 0))

    Off = 128 if Ct <= 128 else 256
    tab_hi = tab.astype(jnp.bfloat16)
    tab_lo = (tab - tab_hi.astype(jnp.float32)).astype(jnp.bfloat16)
    pad = jnp.zeros((B, Np, Off - Ct), jnp.bfloat16)
    tab_hl = jnp.concatenate([tab_hi, pad, tab_lo], axis=-1)

    W1h, W1l = _split_hi_lo(W1)
    W2h, W2l = _split_hi_lo(W2)
    W3h, W3l = _split_hi_lo(W3)

    out = pl.pallas_call(
        functools.partial(_ball_sa_body, Np=Np, K=K, St=St, r2=r2, Ct=Ct,
                          Off=Off),
        grid=grid,
        in_specs=[
            pl.BlockSpec((1, Np, Off + Ct), lambda b, s: (b, 0, 0)),
            pl.BlockSpec((1, 1, Np), lambda b, s: (b, 0, 0)),
            pl.BlockSpec((1, 1, Np), lambda b, s: (b, 0, 0)),
            pl.BlockSpec((1, 1, Np), lambda b, s: (b, 0, 0)),
            pl.BlockSpec((1, St, 1), lambda b, s: (b, s, 0)),
            pl.BlockSpec((1, St, 1), lambda b, s: (b, s, 0)),
            pl.BlockSpec((1, St, 1), lambda b, s: (b, s, 0)),
            c0(L.shape),
            c0(W1.shape), c0(W1.shape), c0((1, W1.shape[1])),
            c0((1, W1.shape[1])), c0((1, W1.shape[1])),
            c0(W2.shape), c0(W2.shape), c0((1, W2.shape[1])),
            c0((1, W2.shape[1])), c0((1, W2.shape[1])),
            c0(W3.shape), c0(W3.shape), c0((1, W3.shape[1])),
            c0((1, W3.shape[1])), c0((1, W3.shape[1])),
        ],
        out_specs=pl.BlockSpec((1, St, Cout), lambda b, s: (b, s, 0)),
        out_shape=jax.ShapeDtypeStruct((B, S, Cout), jnp.float32),
    )(tab_hl,
      Xr.reshape(B, 1, Np), Yr.reshape(B, 1, Np), Zr.reshape(B, 1, Np),
      sx.reshape(B, S, 1), sy.reshape(B, S, 1), sz.reshape(B, S, 1),
      L.astype(jnp.bfloat16),
      W1h, W1l, vec(b1), vec(g1), vec(e1),
      W2h, W2l, vec(b2), vec(g2), vec(e2),
      W3h, W3l, vec(b3), vec(g3), vec(e3))
    return out


def _head_body(t_ref, W1h_ref, W1l_ref, b1_ref, g1_ref, e1_ref, W2h_ref,
               W2l_ref, b2_ref, g2_ref, e2_ref, W3h_ref, W3l_ref, b3_ref,
               g3_ref, e3_ref, f1h_ref, f1l_ref, f1b_ref, n1g_ref, n1b_ref,
               f2h_ref, f2l_ref, f2b_ref, n2g_ref, n2b_ref, f3h_ref, f3l_ref,
               f3b_ref, feat_ref, logp_ref, *, B, Npt):
    h = t_ref[...]
    bnscale = jnp.float32(math.sqrt(1.0 + EPS))
    for Wh, Wl, b, g, e in (
            (W1h_ref, W1l_ref, b1_ref, g1_ref, e1_ref),
            (W2h_ref, W2l_ref, b2_ref, g2_ref, e2_ref),
            (W3h_ref, W3l_ref, b3_ref, g3_ref, e3_ref)):
        h = _dot3(h, Wh[...], Wl[...]) + b[...]
        h = g[...] * h / bnscale + e[...]
        h = jnp.maximum(h, 0.0)
    C = h.shape[1]
    feat = jnp.max(h.reshape(B, Npt, C), axis=1)
    feat_ref[...] = feat

    a = _dot3(feat, f1h_ref[...], f1l_ref[...]) + f1b_ref[...]
    a = n1g_ref[...] * a / bnscale + n1b_ref[...]
    a = jnp.maximum(a, 0.0)
    a = _dot3(a, f2h_ref[...], f2l_ref[...]) + f2b_ref[...]
    a = n2g_ref[...] * a / bnscale + n2b_ref[...]
    a = jnp.maximum(a, 0.0)
    logits = _dot3(a, f3h_ref[...], f3l_ref[...]) + f3b_ref[...]
    mx = jnp.max(logits, axis=1, keepdims=True)
    sh = logits - jax.lax.stop_gradient(mx)
    logp = sh - jnp.log(jnp.sum(jnp.exp(sh), axis=1, keepdims=True))
    logp_ref[...] = logp


def _head(tab, sa3, fc1, bn1, fc2, bn2, fc3, B):
    Npt = tab.shape[0] // B
    (W1, b1, g1, e1), (W2, b2, g2, e2), (W3, b3, g3, e3) = sa3
    vec = lambda v: v.reshape(1, -1)
    nclass = fc3[0].shape[1]
    W1h, W1l = _split_hi_lo(W1)
    W2h, W2l = _split_hi_lo(W2)
    W3h, W3l = _split_hi_lo(W3)
    f1h, f1l = _split_hi_lo(fc1[0])
    f2h, f2l = _split_hi_lo(fc2[0])
    f3h, f3l = _split_hi_lo(fc3[0])
    return pl.pallas_call(
        functools.partial(_head_body, B=B, Npt=Npt),
        out_shape=(jax.ShapeDtypeStruct((B, W3.shape[1]), jnp.float32),
                   jax.ShapeDtypeStruct((B, nclass), jnp.float32)),
    )(tab,
      W1h, W1l, vec(b1), vec(g1), vec(e1),
      W2h, W2l, vec(b2), vec(g2), vec(e2),
      W3h, W3l, vec(b3), vec(g3), vec(e3),
      f1h, f1l, vec(fc1[1]), vec(bn1[0]), vec(bn1[1]),
      f2h, f2l, vec(fc2[1]), vec(bn2[0]), vec(bn2[1]),
      f3h, f3l, vec(fc3[1]))


def _tri(n):
    r = jnp.arange(n, dtype=jnp.int32)
    return (r[:, None] <= r[None, :]).astype(jnp.float32)


def kernel(xyz, params):
    b, t, k, n = xyz.shape
    B = b * t
    pts = xyz.reshape(B, k, n).transpose(0, 2, 1)
    X = pts[:, :, 0]
    Y = pts[:, :, 1]
    Z = pts[:, :, 2]

    n1x, n1y, n1z = _fps(X, Y, Z, 512, pts)
    L1 = _tri(n)
    l1_pts = _ball_sa(pts, X, Y, Z, n1x, n1y, n1z, L1, params['sa1'],
                      r2=0.2 ** 2, K=32, St=256)

    new1 = jnp.stack([n1x, n1y, n1z], axis=-1)
    n2x, n2y, n2z = _fps(n1x, n1y, n1z, 128, new1)
    tab2 = jnp.concatenate([new1, l1_pts], axis=-1)
    L2 = _tri(512)
    l2_pts = _ball_sa(tab2, n1x, n1y, n1z, n2x, n2y, n2z, L2, params['sa2'],
                      r2=0.4 ** 2, K=64, St=64)

    new2 = jnp.stack([n2x, n2y, n2z], axis=-1)
    tab3 = jnp.concatenate([new2, l2_pts], axis=-1).reshape(B * 128, 259)
    feat, logp = _head(tab3, params['sa3'], params['fc1'], params['bn1'],
                       params['fc2'], params['bn2'], params['fc3'], B)

    pred = logp.reshape(b, t, -1).transpose(0, 2, 1)
    features = feat.reshape(b, t, 1024)
    return pred, features


# speedup vs baseline: 10.7783x; 1.0003x over previous
import functools
import math

import jax
import jax.numpy as jnp
from jax import lax
from jax.experimental import pallas as pl
from jax.experimental.pallas import tpu as pltpu
from jax.experimental.pallas import tpu_sc as plsc

EPS = 1e-5


def _dot16(a, b):
    return jnp.dot(a, b, precision=jax.lax.Precision.DEFAULT,
                   preferred_element_type=jnp.float32)


def _split_hi_lo(w):
    hi = w.astype(jnp.bfloat16)
    lo = (w - hi.astype(jnp.float32)).astype(jnp.bfloat16)
    return hi, lo


def _dot3(a, bh, bl):
    ah = a.astype(jnp.bfloat16)
    al = (a - ah.astype(jnp.float32)).astype(jnp.bfloat16)
    return _dot16(ah, bh) + (_dot16(ah, bl) + _dot16(al, bh))


def _fps_body(xyz3_ref, oi_ref, *, S, N, B):
    XYZ3 = xyz3_ref[...]
    X = XYZ3[0:B]
    Y = XYZ3[B:2 * B]
    Z = XYZ3[2 * B:3 * B]
    lid = jax.lax.broadcasted_iota(jnp.int32, (B, N), 1)
    sid = jax.lax.broadcasted_iota(jnp.int32, (B, S), 1)

    def body(i, st):
        dist, idx, aI = st
        aI = aI + (sid == i).astype(jnp.int32) * idx
        oh = (lid == idx).astype(jnp.float32)
        oh3 = jnp.concatenate([oh, oh, oh], axis=0)
        red = jnp.sum(XYZ3 * oh3, axis=1, keepdims=True)
        cx = red[0:B]
        cy = red[B:2 * B]
        cz = red[2 * B:3 * B]
        d = (X - cx) ** 2 + (Y - cy) ** 2 + (Z - cz) ** 2
        dist = jnp.minimum(dist, d)
        m = jnp.max(dist, axis=1, keepdims=True)
        idx = jnp.min(jnp.where(dist == m, lid, N), axis=1, keepdims=True)
        return dist, idx, aI

    dist0 = jnp.full((B, N), 1e10, jnp.float32)
    bidN = jax.lax.broadcasted_iota(jnp.int32, (B, N), 0)
    bidS = jax.lax.broadcasted_iota(jnp.int32, (B, S), 0)
    idx0 = jnp.min(lid * bidN, axis=1, keepdims=True)
    aI0 = jnp.minimum(sid * bidS, 0)
    _, _, aI = jax.lax.fori_loop(0, S, body, (dist0, idx0, aI0))
    oi_ref[...] = aI


def _fps_idx(X, Y, Z, S):
    B, N = X.shape
    xyz3 = jnp.concatenate([X, Y, Z], axis=0)
    return pl.pallas_call(
        functools.partial(_fps_body, S=S, N=N, B=B),
        out_shape=jax.ShapeDtypeStruct((B, S), jnp.int32),
    )(xyz3)


def _sc_gather_rows(idx, pts):
    B, Np, C = pts.shape
    S = idx.shape[1]
    D = 128
    tbl = jnp.concatenate(
        [pts, jnp.zeros((B, Np, D - C), jnp.float32)], axis=-1)
    tbl = tbl.reshape(B * Np, D)
    gidx = (idx + (jnp.arange(B, dtype=jnp.int32) * Np)[:, None]).reshape(-1)
    BT = B * S
    NW = 32
    b_per_w = BT // NW
    mesh = plsc.VectorSubcoreMesh(core_axis_name="c", subcore_axis_name="s")

    @functools.partial(
        pl.kernel, mesh=mesh,
        out_type=jax.ShapeDtypeStruct((BT, D), jnp.float32),
        scratch_types=[
            pltpu.VMEM((b_per_w,), jnp.int32),
            pltpu.VMEM((b_per_w, D), jnp.float32),
            pltpu.SemaphoreType.DMA,
        ],
    )
    def k(table_hbm, idx_hbm, out_hbm, idx_v, rows_v, sem):
        wid = lax.axis_index("s") * 2 + lax.axis_index("c")
        base = wid * b_per_w
        pltpu.sync_copy(idx_hbm.at[pl.ds(base, b_per_w)], idx_v)
        pltpu.async_copy(table_hbm.at[idx_v], rows_v, sem).wait()
        pltpu.sync_copy(rows_v, out_hbm.at[pl.ds(base, b_per_w)])

    return k(tbl, gidx).reshape(B, S, D)


def _fps(X, Y, Z, S, pts):
    idx = _fps_idx(X, Y, Z, S)
    new = _sc_gather_rows(idx, pts)
    return new[:, :, 0], new[:, :, 1], new[:, :, 2]


def _ball_sa_body(tab_ref, xr_ref, yr_ref, zr_ref, sx_ref, sy_ref,
                  sz_ref, L_ref, W1h_ref, W1l_ref, b1_ref, g1_ref, e1_ref,
                  W2h_ref, W2l_ref, b2_ref, g2_ref, e2_ref, W3h_ref, W3l_ref,
                  b3_ref, g3_ref, e3_ref, out_ref, *, Np, K, St, r2, Ct, Off):
    Xr = xr_ref[0]
    Yr = yr_ref[0]
    Zr = zr_ref[0]
    xs = sx_ref[0]
    ys = sy_ref[0]
    zs = sz_ref[0]

    p2 = Xr * Xr + Yr * Yr + Zr * Zr
    s2 = xs * xs + ys * ys + zs * zs
    cross = xs * Xr + ys * Yr + zs * Zr
    d = s2 + p2 - 2.0 * cross
    msk = d <= r2
    mf = msk.astype(jnp.float32).astype(jnp.bfloat16)
    c = _dot16(mf, L_ref[...])
    R = St * K
    c16 = c.astype(jnp.bfloat16)
    cnt3 = jnp.max(c, axis=1, keepdims=True)[:, None, :]
    kio3 = jax.lax.broadcasted_iota(jnp.int32, (St, K, 1), 1).astype(jnp.float32)
    tau3 = jnp.where(kio3 < cnt3, kio3 + 1.0, 1.0).astype(jnp.bfloat16)
    c3 = jnp.broadcast_to(c16[:, None, :], (St, K, Np))
    m3 = jnp.broadcast_to(mf[:, None, :], (St, K, Np))
    Sel = ((c3 == tau3) & (m3 > 0)).astype(jnp.bfloat16).reshape(R, Np)

    Ghl = _dot16(Sel, tab_ref[0])
    G = Ghl[:, 0:Ct] + Ghl[:, Off:Off + Ct]

    lane = jax.lax.broadcasted_iota(jnp.int32, (St, Ct), 1)
    cen = (xs * (lane == 0) + ys * (lane == 1) + zs * (lane == 2))
    cpad = jnp.broadcast_to(cen[:, None, :], (St, K, Ct)).reshape(R, Ct)
    h = G - cpad

    bnscale = jnp.float32(math.sqrt(1.0 + EPS))
    for Wh, Wl, b, g, e in (
            (W1h_ref, W1l_ref, b1_ref, g1_ref, e1_ref),
            (W2h_ref, W2l_ref, b2_ref, g2_ref, e2_ref),
            (W3h_ref, W3l_ref, b3_ref, g3_ref, e3_ref)):
        h = _dot3(h, Wh[...], Wl[...]) + b[...]
        h = g[...] * h / bnscale + e[...]
        h = jnp.maximum(h, 0.0)

    Cout = out_ref.shape[2]
    out_ref[0] = jnp.max(h.reshape(St, K, Cout), axis=1)


def _ball_sa(tab, Xr, Yr, Zr, sx, sy, sz, L, layers, *, r2, K, St):
    B, Np, Ct = tab.shape
    S = sx.shape[1]
    (W1, b1, g1, e1), (W2, b2, g2, e2), (W3, b3, g3, e3) = layers
    Cout = W3.shape[1]
    vec = lambda v: v.reshape(1, -1)
    grid = (B, S // St)
    c0 = lambda shape: pl.BlockSpec(shape, lambda b, s: (0, 0))

    Off = 128 if Ct <= 128 else 256
    tab_hi = tab.astype(jnp.bfloat16)
    tab_lo = (tab - tab_hi.astype(jnp.float32)).astype(jnp.bfloat16)
    pad = jnp.zeros((B, Np, Off - Ct), jnp.bfloat16)
    tab_hl = jnp.concatenate([tab_hi, pad, tab_lo], axis=-1)

    W1h, W1l = _split_hi_lo(W1)
    W2h, W2l = _split_hi_lo(W2)
    W3h, W3l = _split_hi_lo(W3)

    out = pl.pallas_call(
        functools.partial(_ball_sa_body, Np=Np, K=K, St=St, r2=r2, Ct=Ct,
                          Off=Off),
        grid=grid,
        in_specs=[
            pl.BlockSpec((1, Np, Off + Ct), lambda b, s: (b, 0, 0)),
            pl.BlockSpec((1, 1, Np), lambda b, s: (b, 0, 0)),
            pl.BlockSpec((1, 1, Np), lambda b, s: (b, 0, 0)),
            pl.BlockSpec((1, 1, Np), lambda b, s: (b, 0, 0)),
            pl.BlockSpec((1, St, 1), lambda b, s: (b, s, 0)),
            pl.BlockSpec((1, St, 1), lambda b, s: (b, s, 0)),
            pl.BlockSpec((1, St, 1), lambda b, s: (b, s, 0)),
            c0(L.shape),
            c0(W1.shape), c0(W1.shape), c0((1, W1.shape[1])),
            c0((1, W1.shape[1])), c0((1, W1.shape[1])),
            c0(W2.shape), c0(W2.shape), c0((1, W2.shape[1])),
            c0((1, W2.shape[1])), c0((1, W2.shape[1])),
            c0(W3.shape), c0(W3.shape), c0((1, W3.shape[1])),
            c0((1, W3.shape[1])), c0((1, W3.shape[1])),
        ],
        out_specs=pl.BlockSpec((1, St, Cout), lambda b, s: (b, s, 0)),
        out_shape=jax.ShapeDtypeStruct((B, S, Cout), jnp.float32),
    )(tab_hl,
      Xr.reshape(B, 1, Np), Yr.reshape(B, 1, Np), Zr.reshape(B, 1, Np),
      sx.reshape(B, S, 1), sy.reshape(B, S, 1), sz.reshape(B, S, 1),
      L.astype(jnp.bfloat16),
      W1h, W1l, vec(b1), vec(g1), vec(e1),
      W2h, W2l, vec(b2), vec(g2), vec(e2),
      W3h, W3l, vec(b3), vec(g3), vec(e3))
    return out


def _head_body(t_ref, W1h_ref, W1l_ref, b1_ref, g1_ref, e1_ref, W2h_ref,
               W2l_ref, b2_ref, g2_ref, e2_ref, W3h_ref, W3l_ref, b3_ref,
               g3_ref, e3_ref, f1h_ref, f1l_ref, f1b_ref, n1g_ref, n1b_ref,
               f2h_ref, f2l_ref, f2b_ref, n2g_ref, n2b_ref, f3h_ref, f3l_ref,
               f3b_ref, feat_ref, logp_ref, *, B, Npt):
    h = t_ref[...]
    bnscale = jnp.float32(math.sqrt(1.0 + EPS))
    for Wh, Wl, b, g, e in (
            (W1h_ref, W1l_ref, b1_ref, g1_ref, e1_ref),
            (W2h_ref, W2l_ref, b2_ref, g2_ref, e2_ref),
            (W3h_ref, W3l_ref, b3_ref, g3_ref, e3_ref)):
        h = _dot3(h, Wh[...], Wl[...]) + b[...]
        h = g[...] * h / bnscale + e[...]
        h = jnp.maximum(h, 0.0)
    C = h.shape[1]
    feat = jnp.max(h.reshape(B, Npt, C), axis=1)
    feat_ref[...] = feat

    a = _dot3(feat, f1h_ref[...], f1l_ref[...]) + f1b_ref[...]
    a = n1g_ref[...] * a / bnscale + n1b_ref[...]
    a = jnp.maximum(a, 0.0)
    a = _dot3(a, f2h_ref[...], f2l_ref[...]) + f2b_ref[...]
    a = n2g_ref[...] * a / bnscale + n2b_ref[...]
    a = jnp.maximum(a, 0.0)
    logits = _dot3(a, f3h_ref[...], f3l_ref[...]) + f3b_ref[...]
    mx = jnp.max(logits, axis=1, keepdims=True)
    sh = logits - jax.lax.stop_gradient(mx)
    logp = sh - jnp.log(jnp.sum(jnp.exp(sh), axis=1, keepdims=True))
    logp_ref[...] = logp


def _head(tab, sa3, fc1, bn1, fc2, bn2, fc3, B):
    Npt = tab.shape[0] // B
    (W1, b1, g1, e1), (W2, b2, g2, e2), (W3, b3, g3, e3) = sa3
    vec = lambda v: v.reshape(1, -1)
    nclass = fc3[0].shape[1]
    W1h, W1l = _split_hi_lo(W1)
    W2h, W2l = _split_hi_lo(W2)
    W3h, W3l = _split_hi_lo(W3)
    f1h, f1l = _split_hi_lo(fc1[0])
    f2h, f2l = _split_hi_lo(fc2[0])
    f3h, f3l = _split_hi_lo(fc3[0])
    return pl.pallas_call(
        functools.partial(_head_body, B=B, Npt=Npt),
        out_shape=(jax.ShapeDtypeStruct((B, W3.shape[1]), jnp.float32),
                   jax.ShapeDtypeStruct((B, nclass), jnp.float32)),
    )(tab,
      W1h, W1l, vec(b1), vec(g1), vec(e1),
      W2h, W2l, vec(b2), vec(g2), vec(e2),
      W3h, W3l, vec(b3), vec(g3), vec(e3),
      f1h, f1l, vec(fc1[1]), vec(bn1[0]), vec(bn1[1]),
      f2h, f2l, vec(fc2[1]), vec(bn2[0]), vec(bn2[1]),
      f3h, f3l, vec(fc3[1]))


def _tri(n):
    r = jnp.arange(n, dtype=jnp.int32)
    return (r[:, None] <= r[None, :]).astype(jnp.float32)


def kernel(xyz, params):
    b, t, k, n = xyz.shape
    B = b * t
    pts = xyz.reshape(B, k, n).transpose(0, 2, 1)
    X = pts[:, :, 0]
    Y = pts[:, :, 1]
    Z = pts[:, :, 2]

    n1x, n1y, n1z = _fps(X, Y, Z, 512, pts)
    L1 = _tri(n)
    l1_pts = _ball_sa(pts, X, Y, Z, n1x, n1y, n1z, L1, params['sa1'],
                      r2=0.2 ** 2, K=32, St=256)

    new1 = jnp.stack([n1x, n1y, n1z], axis=-1)
    n2x, n2y, n2z = _fps(n1x, n1y, n1z, 128, new1)
    tab2 = jnp.concatenate([new1, l1_pts], axis=-1)
    L2 = _tri(512)
    l2_pts = _ball_sa(tab2, n1x, n1y, n1z, n2x, n2y, n2z, L2, params['sa2'],
                      r2=0.4 ** 2, K=64, St=64)

    new2 = jnp.stack([n2x, n2y, n2z], axis=-1)
    tab3 = jnp.concatenate([new2, l2_pts], axis=-1).reshape(B * 128, 259)
    feat, logp = _head(tab3, params['sa3'], params['fc1'], params['bn1'],
                       params['fc2'], params['bn2'], params['fc3'], B)

    pred = logp.reshape(b, t, -1).transpose(0, 2, 1)
    features = feat.reshape(b, t, 1024)
    return pred, features


# speedup vs baseline: 10.8756x; 1.0090x over previous
import functools
import math

import jax
import jax.numpy as jnp
from jax import lax
from jax.experimental import pallas as pl
from jax.experimental.pallas import tpu as pltpu
from jax.experimental.pallas import tpu_sc as plsc

EPS = 1e-5


def _dot16(a, b):
    return jnp.dot(a, b, precision=jax.lax.Precision.DEFAULT,
                   preferred_element_type=jnp.float32)


def _split_hi_lo(w):
    hi = w.astype(jnp.bfloat16)
    lo = (w - hi.astype(jnp.float32)).astype(jnp.bfloat16)
    return hi, lo


def _dot3(a, bh, bl):
    ah = a.astype(jnp.bfloat16)
    al = (a - ah.astype(jnp.float32)).astype(jnp.bfloat16)
    return _dot16(ah, bh) + (_dot16(ah, bl) + _dot16(al, bh))


def _fps_body(xyz3_ref, oi_ref, *, S, N, B):
    XYZ3 = xyz3_ref[...]
    X = XYZ3[0:B]
    Y = XYZ3[B:2 * B]
    Z = XYZ3[2 * B:3 * B]
    lid = jax.lax.broadcasted_iota(jnp.int32, (B, N), 1)
    sid = jax.lax.broadcasted_iota(jnp.int32, (B, S), 1)

    def body(i, st):
        dist, idx, aI = st
        aI = aI + (sid == i).astype(jnp.int32) * idx
        oh = (lid == idx).astype(jnp.float32)
        oh3 = jnp.concatenate([oh, oh, oh], axis=0)
        red = jnp.sum(XYZ3 * oh3, axis=1, keepdims=True)
        cx = red[0:B]
        cy = red[B:2 * B]
        cz = red[2 * B:3 * B]
        d = (X - cx) ** 2 + (Y - cy) ** 2 + (Z - cz) ** 2
        dist = jnp.minimum(dist, d)
        m = jnp.max(dist, axis=1, keepdims=True)
        idx = jnp.min(jnp.where(dist == m, lid, N), axis=1, keepdims=True)
        return dist, idx, aI

    dist0 = jnp.full((B, N), 1e10, jnp.float32)
    bidN = jax.lax.broadcasted_iota(jnp.int32, (B, N), 0)
    bidS = jax.lax.broadcasted_iota(jnp.int32, (B, S), 0)
    idx0 = jnp.min(lid * bidN, axis=1, keepdims=True)
    aI0 = jnp.minimum(sid * bidS, 0)
    _, _, aI = jax.lax.fori_loop(0, S, body, (dist0, idx0, aI0))
    oi_ref[...] = aI


def _fps_idx(X, Y, Z, S):
    B, N = X.shape
    xyz3 = jnp.concatenate([X, Y, Z], axis=0)
    return pl.pallas_call(
        functools.partial(_fps_body, S=S, N=N, B=B),
        out_shape=jax.ShapeDtypeStruct((B, S), jnp.int32),
    )(xyz3)


def _sc_gather_rows(idx, pts):
    B, Np, C = pts.shape
    S = idx.shape[1]
    D = 128
    tbl = jnp.concatenate(
        [pts, jnp.zeros((B, Np, D - C), jnp.float32)], axis=-1)
    tbl = tbl.reshape(B * Np, D)
    gidx = (idx + (jnp.arange(B, dtype=jnp.int32) * Np)[:, None]).reshape(-1)
    BT = B * S
    NW = 32
    b_per_w = BT // NW
    mesh = plsc.VectorSubcoreMesh(core_axis_name="c", subcore_axis_name="s")

    @functools.partial(
        pl.kernel, mesh=mesh,
        out_type=jax.ShapeDtypeStruct((BT, D), jnp.float32),
        scratch_types=[
            pltpu.VMEM((b_per_w,), jnp.int32),
            pltpu.VMEM((b_per_w, D), jnp.float32),
            pltpu.SemaphoreType.DMA,
        ],
    )
    def k(table_hbm, idx_hbm, out_hbm, idx_v, rows_v, sem):
        wid = lax.axis_index("s") * 2 + lax.axis_index("c")
        base = wid * b_per_w
        pltpu.sync_copy(idx_hbm.at[pl.ds(base, b_per_w)], idx_v)
        pltpu.async_copy(table_hbm.at[idx_v], rows_v, sem).wait()
        pltpu.sync_copy(rows_v, out_hbm.at[pl.ds(base, b_per_w)])

    return k(tbl, gidx).reshape(B, S, D)


def _fps(X, Y, Z, S, pts):
    idx = _fps_idx(X, Y, Z, S)
    new = _sc_gather_rows(idx, pts)
    return new[:, :, 0], new[:, :, 1], new[:, :, 2]


def _ball_sa_body(tab_ref, xr_ref, yr_ref, zr_ref, sx_ref, sy_ref,
                  sz_ref, L_ref, W1h_ref, W1l_ref, b1_ref, g1_ref, e1_ref,
                  W2h_ref, W2l_ref, b2_ref, g2_ref, e2_ref, W3h_ref, W3l_ref,
                  b3_ref, g3_ref, e3_ref, out_ref, *, Np, K, St, r2, Ct, Off):
    Xr = xr_ref[0]
    Yr = yr_ref[0]
    Zr = zr_ref[0]
    xs = sx_ref[0]
    ys = sy_ref[0]
    zs = sz_ref[0]

    p2 = Xr * Xr + Yr * Yr + Zr * Zr
    s2 = xs * xs + ys * ys + zs * zs
    cross = xs * Xr + ys * Yr + zs * Zr
    d = s2 + p2 - 2.0 * cross
    msk = d <= r2
    mf = msk.astype(jnp.float32).astype(jnp.bfloat16)
    c = _dot16(mf, L_ref[...])
    R = St * K
    cq16 = (c * mf.astype(jnp.float32)).astype(jnp.bfloat16)
    cnt3 = jnp.max(c, axis=1, keepdims=True)[:, None, :]
    kio3 = jax.lax.broadcasted_iota(jnp.int32, (St, K, 1), 1).astype(jnp.float32)
    tau3 = jnp.where(kio3 < cnt3, kio3 + 1.0, 1.0).astype(jnp.bfloat16)
    cq3 = jnp.broadcast_to(cq16[:, None, :], (St, K, Np))
    Sel = (cq3 == tau3).astype(jnp.bfloat16).reshape(R, Np)

    Ghl = _dot16(Sel, tab_ref[0])
    G = Ghl[:, 0:Ct] + Ghl[:, Off:Off + Ct]

    lane = jax.lax.broadcasted_iota(jnp.int32, (St, Ct), 1)
    cen = (xs * (lane == 0) + ys * (lane == 1) + zs * (lane == 2))
    cpad = jnp.broadcast_to(cen[:, None, :], (St, K, Ct)).reshape(R, Ct)
    h = G - cpad

    for Wh, Wl, b, g, e in (
            (W1h_ref, W1l_ref, b1_ref, g1_ref, e1_ref),
            (W2h_ref, W2l_ref, b2_ref, g2_ref, e2_ref),
            (W3h_ref, W3l_ref, b3_ref, g3_ref, e3_ref)):
        h = _dot3(h, Wh[...], Wl[...]) + b[...]
        h = g[...] * h + e[...]
        h = jnp.maximum(h, 0.0)

    Cout = out_ref.shape[2]
    out_ref[0] = jnp.max(h.reshape(St, K, Cout), axis=1)


def _ball_sa(tab, Xr, Yr, Zr, sx, sy, sz, L, layers, *, r2, K, St):
    B, Np, Ct = tab.shape
    S = sx.shape[1]
    (W1, b1, g1, e1), (W2, b2, g2, e2), (W3, b3, g3, e3) = layers
    Cout = W3.shape[1]
    vec = lambda v: v.reshape(1, -1)
    vecg = lambda v: (v / jnp.float32(math.sqrt(1.0 + EPS))).reshape(1, -1)
    grid = (B, S // St)
    c0 = lambda shape: pl.BlockSpec(shape, lambda b, s: (0, 0))

    Off = 128 if Ct <= 128 else 256
    tab_hi = tab.astype(jnp.bfloat16)
    tab_lo = (tab - tab_hi.astype(jnp.float32)).astype(jnp.bfloat16)
    pad = jnp.zeros((B, Np, Off - Ct), jnp.bfloat16)
    tab_hl = jnp.concatenate([tab_hi, pad, tab_lo], axis=-1)

    W1h, W1l = _split_hi_lo(W1)
    W2h, W2l = _split_hi_lo(W2)
    W3h, W3l = _split_hi_lo(W3)

    out = pl.pallas_call(
        functools.partial(_ball_sa_body, Np=Np, K=K, St=St, r2=r2, Ct=Ct,
                          Off=Off),
        grid=grid,
        in_specs=[
            pl.BlockSpec((1, Np, Off + Ct), lambda b, s: (b, 0, 0)),
            pl.BlockSpec((1, 1, Np), lambda b, s: (b, 0, 0)),
            pl.BlockSpec((1, 1, Np), lambda b, s: (b, 0, 0)),
            pl.BlockSpec((1, 1, Np), lambda b, s: (b, 0, 0)),
            pl.BlockSpec((1, St, 1), lambda b, s: (b, s, 0)),
            pl.BlockSpec((1, St, 1), lambda b, s: (b, s, 0)),
            pl.BlockSpec((1, St, 1), lambda b, s: (b, s, 0)),
            c0(L.shape),
            c0(W1.shape), c0(W1.shape), c0((1, W1.shape[1])),
            c0((1, W1.shape[1])), c0((1, W1.shape[1])),
            c0(W2.shape), c0(W2.shape), c0((1, W2.shape[1])),
            c0((1, W2.shape[1])), c0((1, W2.shape[1])),
            c0(W3.shape), c0(W3.shape), c0((1, W3.shape[1])),
            c0((1, W3.shape[1])), c0((1, W3.shape[1])),
        ],
        out_specs=pl.BlockSpec((1, St, Cout), lambda b, s: (b, s, 0)),
        out_shape=jax.ShapeDtypeStruct((B, S, Cout), jnp.float32),
    )(tab_hl,
      Xr.reshape(B, 1, Np), Yr.reshape(B, 1, Np), Zr.reshape(B, 1, Np),
      sx.reshape(B, S, 1), sy.reshape(B, S, 1), sz.reshape(B, S, 1),
      L.astype(jnp.bfloat16),
      W1h, W1l, vec(b1), vecg(g1), vec(e1),
      W2h, W2l, vec(b2), vecg(g2), vec(e2),
      W3h, W3l, vec(b3), vecg(g3), vec(e3))
    return out


def _head_body(t_ref, W1h_ref, W1l_ref, b1_ref, g1_ref, e1_ref, W2h_ref,
               W2l_ref, b2_ref, g2_ref, e2_ref, W3h_ref, W3l_ref, b3_ref,
               g3_ref, e3_ref, f1h_ref, f1l_ref, f1b_ref, n1g_ref, n1b_ref,
               f2h_ref, f2l_ref, f2b_ref, n2g_ref, n2b_ref, f3h_ref, f3l_ref,
               f3b_ref, feat_ref, logp_ref, *, B, Npt):
    h = t_ref[...]
    for Wh, Wl, b, g, e in (
            (W1h_ref, W1l_ref, b1_ref, g1_ref, e1_ref),
            (W2h_ref, W2l_ref, b2_ref, g2_ref, e2_ref),
            (W3h_ref, W3l_ref, b3_ref, g3_ref, e3_ref)):
        h = _dot3(h, Wh[...], Wl[...]) + b[...]
        h = g[...] * h + e[...]
        h = jnp.maximum(h, 0.0)
    C = h.shape[1]
    feat = jnp.max(h.reshape(B, Npt, C), axis=1)
    feat_ref[...] = feat

    a = _dot3(feat, f1h_ref[...], f1l_ref[...]) + f1b_ref[...]
    a = n1g_ref[...] * a + n1b_ref[...]
    a = jnp.maximum(a, 0.0)
    a = _dot3(a, f2h_ref[...], f2l_ref[...]) + f2b_ref[...]
    a = n2g_ref[...] * a + n2b_ref[...]
    a = jnp.maximum(a, 0.0)
    logits = _dot3(a, f3h_ref[...], f3l_ref[...]) + f3b_ref[...]
    mx = jnp.max(logits, axis=1, keepdims=True)
    sh = logits - jax.lax.stop_gradient(mx)
    logp = sh - jnp.log(jnp.sum(jnp.exp(sh), axis=1, keepdims=True))
    logp_ref[...] = logp


def _head(tab, sa3, fc1, bn1, fc2, bn2, fc3, B):
    Npt = tab.shape[0] // B
    (W1, b1, g1, e1), (W2, b2, g2, e2), (W3, b3, g3, e3) = sa3
    vec = lambda v: v.reshape(1, -1)
    vecg = lambda v: (v / jnp.float32(math.sqrt(1.0 + EPS))).reshape(1, -1)
    nclass = fc3[0].shape[1]
    W1h, W1l = _split_hi_lo(W1)
    W2h, W2l = _split_hi_lo(W2)
    W3h, W3l = _split_hi_lo(W3)
    f1h, f1l = _split_hi_lo(fc1[0])
    f2h, f2l = _split_hi_lo(fc2[0])
    f3h, f3l = _split_hi_lo(fc3[0])
    return pl.pallas_call(
        functools.partial(_head_body, B=B, Npt=Npt),
        out_shape=(jax.ShapeDtypeStruct((B, W3.shape[1]), jnp.float32),
                   jax.ShapeDtypeStruct((B, nclass), jnp.float32)),
    )(tab,
      W1h, W1l, vec(b1), vecg(g1), vec(e1),
      W2h, W2l, vec(b2), vecg(g2), vec(e2),
      W3h, W3l, vec(b3), vecg(g3), vec(e3),
      f1h, f1l, vec(fc1[1]), vecg(bn1[0]), vec(bn1[1]),
      f2h, f2l, vec(fc2[1]), vecg(bn2[0]), vec(bn2[1]),
      f3h, f3l, vec(fc3[1]))


def _tri(n):
    r = jnp.arange(n, dtype=jnp.int32)
    return (r[:, None] <= r[None, :]).astype(jnp.float32)


def kernel(xyz, params):
    b, t, k, n = xyz.shape
    B = b * t
    pts = xyz.reshape(B, k, n).transpose(0, 2, 1)
    X = pts[:, :, 0]
    Y = pts[:, :, 1]
    Z = pts[:, :, 2]

    n1x, n1y, n1z = _fps(X, Y, Z, 512, pts)
    L1 = _tri(n)
    l1_pts = _ball_sa(pts, X, Y, Z, n1x, n1y, n1z, L1, params['sa1'],
                      r2=0.2 ** 2, K=32, St=256)

    new1 = jnp.stack([n1x, n1y, n1z], axis=-1)
    n2x, n2y, n2z = _fps(n1x, n1y, n1z, 128, new1)
    tab2 = jnp.concatenate([new1, l1_pts], axis=-1)
    L2 = _tri(512)
    l2_pts = _ball_sa(tab2, n1x, n1y, n1z, n2x, n2y, n2z, L2, params['sa2'],
                      r2=0.4 ** 2, K=64, St=64)

    new2 = jnp.stack([n2x, n2y, n2z], axis=-1)
    tab3 = jnp.concatenate([new2, l2_pts], axis=-1).reshape(B * 128, 259)
    feat, logp = _head(tab3, params['sa3'], params['fc1'], params['bn1'],
                       params['fc2'], params['bn2'], params['fc3'], B)

    pred = logp.reshape(b, t, -1).transpose(0, 2, 1)
    features = feat.reshape(b, t, 1024)
    return pred, features
